# baseline (speedup 1.0000x reference)
import functools

import jax
import jax.numpy as jnp
from jax import lax
from jax.experimental import pallas as pl
from jax.experimental.pallas import tpu as pltpu
from jax.experimental.pallas import tpu_sc as plsc

D_MODEL = 64
NC, NS = 2, 16
NW = NC * NS
LANES = 128
NBUF = 4


@functools.lru_cache(maxsize=None)
def _build(t_dim: int, d: int):
    mesh = plsc.VectorSubcoreMesh(core_axis_name="c", subcore_axis_name="s")
    n_outer = t_dim // NBUF
    dk = d // 8
    hw = d // 2

    @functools.partial(
        pl.kernel,
        out_type=jax.ShapeDtypeStruct((t_dim, dk, NW, 8, LANES), jnp.float32),
        mesh=mesh,
        scratch_types=[
            pltpu.VMEM((t_dim, LANES), jnp.int32),
            [(pltpu.VMEM((LANES, hw), jnp.float32),
              pltpu.VMEM((LANES, hw), jnp.float32))] * NBUF,
            [pltpu.VMEM((dk, 8, LANES + 1), jnp.float32)] * NBUF,
            pltpu.SemaphoreType.DMA,
            [pltpu.SemaphoreType.DMA] * NBUF,
            [pltpu.SemaphoreType.DMA] * NBUF,
        ],
        compiler_params=pltpu.CompilerParams(
            use_tc_tiling_on_sc=False, needs_layout_passes=False),
    )
    def emb_kernel(xt_hbm, th0_hbm, th1_hbm, out_hbm, idx_v, gbufs, obufs,
                   isem, gsems, osems):
        wid = lax.axis_index("s") * NC + lax.axis_index("c")
        pltpu.async_copy(
            xt_hbm.at[:, pl.ds(wid * LANES, LANES)], idx_v, isem).wait()

        halves = (th0_hbm, th1_hbm)

        def fire_gathers(t, b):
            for h in range(2):
                pltpu.async_copy(
                    halves[h].at[idx_v.at[t]], gbufs[b][h], gsems[b])

        def wait_gathers(t, b):
            for h in range(2):
                pltpu.make_async_copy(
                    halves[h].at[idx_v.at[t]], gbufs[b][h], gsems[b]).wait()

        for b in range(NBUF):
            fire_gathers(b, b)

        iota16 = lax.iota(jnp.int32, 16)
        zero16 = jnp.zeros((16,), jnp.int32)
        kvecs = [(iota16 + c0) // 8 for c0 in range(0, d, 16)]
        svecs = [(iota16 + c0) % 8 for c0 in range(0, d, 16)]
        scale = float(d) ** 0.5
        UNR = 8

        def outer(go, carry):
            for b in range(NBUF):
                t = go * NBUF + b
                gbuf, obuf = gbufs[b], obufs[b]
                wait_gathers(t, b)

                @pl.when(go > 0)
                def _():
                    pltpu.make_async_copy(
                        obuf.at[:, :, pl.ds(0, LANES)],
                        out_hbm.at[t, :, wid], osems[b]).wait()

                def rbody(ro, c2):
                    base = ro * UNR
                    for j in range(UNR):
                        r = base + j
                        colv = zero16 + r
                        for ci in range(d // 16):
                            src = gbuf[ci // (hw // 16)]
                            co = ci % (hw // 16)
                            vec = src[r, pl.ds(16 * co, 16)] * scale
                            plsc.store_scatter(
                                obuf, [kvecs[ci], svecs[ci], colv], vec)
                    return c2

                lax.fori_loop(0, LANES // UNR, rbody, 0)

                @pl.when(go < n_outer - 1)
                def _():
                    fire_gathers(t + NBUF, b)

                pltpu.async_copy(
                    obuf.at[:, :, pl.ds(0, LANES)],
                    out_hbm.at[t, :, wid], osems[b])
            return carry

        lax.fori_loop(0, n_outer, outer, 0)

        for b in range(NBUF):
            t = t_dim - NBUF + b
            pltpu.make_async_copy(
                obufs[b].at[:, :, pl.ds(0, LANES)],
                out_hbm.at[t, :, wid], osems[b]).wait()

    return emb_kernel


@jax.jit
def kernel(x, table):
    bsz, t_dim = x.shape
    v, d = table.shape
    assert bsz == NW * LANES and d % 16 == 0 and t_dim % NBUF == 0
    xt = x.T.astype(jnp.int32)
    th0 = table[:, :d // 2]
    th1 = table[:, d // 2:]
    out5 = _build(t_dim, d)(xt, th0, th1)
    return out5.transpose(2, 4, 0, 1, 3).reshape(bsz, t_dim, d)


# speedup vs baseline: 2.5310x; 2.5310x over previous
import functools

import jax
import jax.numpy as jnp
from jax import lax
from jax.experimental import pallas as pl
from jax.experimental.pallas import tpu as pltpu
from jax.experimental.pallas import tpu_sc as plsc

D_MODEL = 64
NC, NS = 2, 16
NW = NC * NS
LANES = 128
NBUF = 4


@functools.lru_cache(maxsize=None)
def _build(t_dim: int, d: int):
    mesh = plsc.VectorSubcoreMesh(core_axis_name="c", subcore_axis_name="s")
    n_outer = t_dim // NBUF
    dk = d // 8
    hw = d // 2

    @functools.partial(
        pl.kernel,
        out_type=jax.ShapeDtypeStruct((t_dim, dk, NW, 8, LANES), jnp.float32),
        mesh=mesh,
        scratch_types=[
            pltpu.VMEM((t_dim, LANES), jnp.int32),
            [pltpu.VMEM((LANES, d), jnp.float32)] * NBUF,
            [pltpu.VMEM((dk, 8, LANES + 1), jnp.float32)] * NBUF,
            pltpu.SemaphoreType.DMA,
            [pltpu.SemaphoreType.DMA] * NBUF,
            [pltpu.SemaphoreType.DMA] * NBUF,
        ],
        compiler_params=pltpu.CompilerParams(
            use_tc_tiling_on_sc=False, needs_layout_passes=False),
    )
    def emb_kernel(xt_hbm, tbl_hbm, out_hbm, idx_v, gbufs, obufs,
                   isem, gsems, osems):
        wid = lax.axis_index("s") * NC + lax.axis_index("c")
        pltpu.async_copy(
            xt_hbm.at[:, pl.ds(wid * LANES, LANES)], idx_v, isem).wait()

        def fire_gathers(t, b):
            pltpu.async_copy(tbl_hbm.at[idx_v.at[t]], gbufs[b], gsems[b])

        def wait_gathers(t, b):
            pltpu.make_async_copy(
                tbl_hbm.at[idx_v.at[t]], gbufs[b], gsems[b]).wait()

        for b in range(NBUF):
            fire_gathers(b, b)

        iota16 = lax.iota(jnp.int32, 16)
        zero16 = jnp.zeros((16,), jnp.int32)
        kvecs = [(iota16 + c0) // 8 for c0 in range(0, d, 16)]
        svecs = [(iota16 + c0) % 8 for c0 in range(0, d, 16)]
        scale = float(d) ** 0.5
        UNR = 8

        def outer(go, carry):
            for b in range(NBUF):
                t = go * NBUF + b
                gbuf, obuf = gbufs[b], obufs[b]
                wait_gathers(t, b)

                @pl.when(go > 0)
                def _():
                    pltpu.make_async_copy(
                        obuf.at[:, :, pl.ds(0, LANES)],
                        out_hbm.at[t, :, wid], osems[b]).wait()

                @plsc.parallel_loop(0, LANES, 1, unroll=4)
                def rbody(r):
                    colv = zero16 + r
                    for ci in range(d // 16):
                        vec = gbuf[r, pl.ds(16 * ci, 16)] * scale
                        plsc.store_scatter(
                            obuf, [kvecs[ci], svecs[ci], colv], vec)

                @pl.when(go < n_outer - 1)
                def _():
                    fire_gathers(t + NBUF, b)

                pltpu.async_copy(
                    obuf.at[:, :, pl.ds(0, LANES)],
                    out_hbm.at[t, :, wid], osems[b])
            return carry

        lax.fori_loop(0, n_outer, outer, 0)

        for b in range(NBUF):
            t = t_dim - NBUF + b
            pltpu.make_async_copy(
                obufs[b].at[:, :, pl.ds(0, LANES)],
                out_hbm.at[t, :, wid], osems[b]).wait()

    return emb_kernel


@jax.jit
def kernel(x, table):
    bsz, t_dim = x.shape
    v, d = table.shape
    assert bsz == NW * LANES and d % 16 == 0 and t_dim % NBUF == 0
    xt = x.T.astype(jnp.int32)
    out5 = _build(t_dim, d)(xt, table)
    return out5.transpose(2, 4, 0, 1, 3).reshape(bsz, t_dim, d)
